# baseline (speedup 1.0000x reference)
import functools

import jax
import jax.numpy as jnp
from jax import lax
from jax.experimental import pallas as pl
from jax.experimental.pallas import tpu as pltpu
from jax.experimental.pallas import tpu_sc as plsc

_NV = 10000
_NC = 42000
_K = 3
_B = 128
_NSUB = 16
_NW = 32
_CPW = _NC // _NW
_CPW_LAST = _NC - (_NW - 1) * _CPW
_G = _CPW // 16
_G_LAST = _CPW_LAST // 16
_THRESH = float(_NC - 1)


def _sat_body(emb_hbm, lit_hbm, out_hbm,
              table_v, l0, l1, l2, part_v, part_sh, all_v,
              sem_t, sem_w):
    cid = lax.axis_index("c")
    sid = lax.axis_index("s")
    wid = cid * _NSUB + sid
    start = wid * _CPW
    last = wid == (_NW - 1)
    bufs = (l0, l1, l2)
    cp_t = pltpu.make_async_copy(emb_hbm.at[0], table_v, sem_t)
    cp_t.start()

    @pl.when(jnp.logical_not(last))
    def _copy_most():
        for j in range(_K):
            off = pl.multiple_of(j * _NC + start, 8)
            pltpu.make_async_copy(lit_hbm.at[pl.ds(off, _CPW)],
                                  bufs[j].at[pl.ds(0, _CPW)], sem_w).start()
        for j in range(_K):
            pltpu.make_async_copy(lit_hbm.at[pl.ds(0, _CPW)],
                                  bufs[j].at[pl.ds(0, _CPW)], sem_w).wait()

    @pl.when(last)
    def _copy_last():
        for j in range(_K):
            off = pl.multiple_of(j * _NC + start, 8)
            pltpu.make_async_copy(lit_hbm.at[pl.ds(off, _CPW_LAST)],
                                  bufs[j], sem_w).start()
        for j in range(_K):
            pltpu.make_async_copy(lit_hbm.at[pl.ds(0, _CPW_LAST)],
                                  bufs[j], sem_w).wait()

    cp_t.wait()

    def body(i, acc):
        base = i * 16
        sat = None
        for j in range(_K):
            p = bufs[j][pl.ds(base, 16)]
            lit = lax.shift_right_logical(p, 1)
            ev = plsc.load_gather(table_v, [lit])
            t = (ev < 0.0) == ((p & 1) == 1)
            sat = t if sat is None else jnp.logical_or(sat, t)
        return acc + jnp.where(sat, 1.0, -1.0)

    ngroups = jnp.where(last, _G_LAST, _G)
    acc = lax.fori_loop(0, ngroups, body, jnp.zeros((16,), jnp.float32))

    part_v[...] = acc
    pltpu.sync_copy(part_v, part_sh.at[sid])
    plsc.subcore_barrier()

    @pl.when(sid == 0)
    def _finish():
        pltpu.sync_copy(part_sh, all_v)
        tot = all_v[0]
        for r in range(1, _NSUB):
            tot = tot + all_v[r]
        part_v[...] = tot
        pltpu.sync_copy(part_v, out_hbm.at[cid])


@functools.lru_cache(maxsize=1)
def _build():
    mesh = plsc.VectorSubcoreMesh(
        core_axis_name="c", subcore_axis_name="s",
        num_cores=2, num_subcores=_NSUB,
    )
    return pl.kernel(
        _sat_body,
        out_type=jax.ShapeDtypeStruct((2, 16), jnp.float32),
        mesh=mesh,
        compiler_params=pltpu.CompilerParams(needs_layout_passes=False),
        scratch_types=[
            pltpu.VMEM((_NV,), jnp.float32),
            pltpu.VMEM((_CPW_LAST,), jnp.int32),
            pltpu.VMEM((_CPW_LAST,), jnp.int32),
            pltpu.VMEM((_CPW_LAST,), jnp.int32),
            pltpu.VMEM((16,), jnp.float32),
            pltpu.VMEM_SHARED((_NSUB, 16), jnp.float32),
            pltpu.VMEM((_NSUB, 16), jnp.float32),
            pltpu.SemaphoreType.DMA,
            pltpu.SemaphoreType.DMA,
        ],
    )


def kernel(input, emb_weight, or_weight, clause_idx):
    del input
    packed = lax.shift_left(clause_idx, 1) | (or_weight < 0).astype(jnp.int32)
    parts = _build()(emb_weight, packed.T.reshape(-1))
    return jnp.broadcast_to(jnp.sign(jnp.sum(parts) - _THRESH), (_B,))


# speedup vs baseline: 1.0719x; 1.0719x over previous
import functools

import jax
import jax.numpy as jnp
from jax import lax
from jax.experimental import pallas as pl
from jax.experimental.pallas import tpu as pltpu
from jax.experimental.pallas import tpu_sc as plsc

_NV = 10000
_NC = 42000
_K = 3
_B = 128
_NSUB = 16
_CPW = _NC // _NSUB
_FULL = _CPW // 16
_REM = _CPW - _FULL * 16
_WIN = 2640
_WIN_LAST = 2632
_BUF = 2648
_THRESH = float(_NC - 1 + _NSUB * (16 - _REM))


def _sat_body(emb_hbm, lit_hbm, out_hbm,
              table_v, l0, l1, l2, part_v, part_sh, all_v, out_v,
              sem_t, sem_w):
    cid = lax.axis_index("c")
    sid = lax.axis_index("s")
    start = sid * _CPW
    astart = (start // 8) * 8
    delta = start - astart
    bufs = (l0, l1, l2)
    cp_t = pltpu.make_async_copy(emb_hbm.at[0], table_v, sem_t)
    cp_t.start()

    @pl.when(sid < 15)
    def _copy_most():
        for j in range(_K):
            off = pl.multiple_of(j * _NC + astart, 8)
            pltpu.make_async_copy(lit_hbm.at[pl.ds(off, _WIN)],
                                  bufs[j].at[pl.ds(0, _WIN)], sem_w).start()
        for j in range(_K):
            pltpu.make_async_copy(lit_hbm.at[pl.ds(0, _WIN)],
                                  bufs[j].at[pl.ds(0, _WIN)], sem_w).wait()

    @pl.when(sid == 15)
    def _copy_last():
        for j in range(_K):
            off = pl.multiple_of(j * _NC + astart, 8)
            pltpu.make_async_copy(lit_hbm.at[pl.ds(off, _WIN_LAST)],
                                  bufs[j].at[pl.ds(0, _WIN_LAST)],
                                  sem_w).start()
        for j in range(_K):
            pltpu.make_async_copy(lit_hbm.at[pl.ds(0, _WIN_LAST)],
                                  bufs[j].at[pl.ds(0, _WIN_LAST)],
                                  sem_w).wait()

    cp_t.wait()
    lanes = lax.iota(jnp.int32, 16)

    def body(i, acc):
        base = delta + i * 16
        sat = None
        for j in range(_K):
            p = bufs[j][pl.ds(base, 16)]
            lit = lax.shift_right_logical(p, 1)
            ev = plsc.load_gather(table_v, [lit])
            t = (ev < 0.0) == ((p & 1) == 1)
            sat = t if sat is None else jnp.logical_or(sat, t)
        return acc + jnp.where(sat, 1.0, -1.0)

    acc = lax.fori_loop(0, _FULL, body, jnp.zeros((16,), jnp.float32))

    valid = lanes < _REM
    base = delta + _FULL * 16
    sat = None
    for j in range(_K):
        p = bufs[j][pl.ds(base, 16)]
        lit = jnp.where(valid, lax.shift_right_logical(p, 1), 0)
        ev = plsc.load_gather(table_v, [lit])
        t = (ev < 0.0) == ((p & 1) == 1)
        sat = t if sat is None else jnp.logical_or(sat, t)
    acc = acc + jnp.where(jnp.logical_or(sat, jnp.logical_not(valid)),
                          1.0, -1.0)

    part_v[...] = acc
    pltpu.sync_copy(part_v, part_sh.at[sid])
    plsc.subcore_barrier()

    @pl.when(jnp.logical_and(cid == 0, sid == 0))
    def _finish():
        pltpu.sync_copy(part_sh, all_v)
        tot = all_v[0]
        for r in range(1, _NSUB):
            tot = tot + all_v[r]
        total = jnp.sum(tot)
        res = jnp.sign(total - _THRESH)
        resv = jnp.full((16,), res, jnp.float32)
        for k in range(_B // 16):
            out_v[pl.ds(k * 16, 16)] = resv
        pltpu.sync_copy(out_v, out_hbm)


@functools.lru_cache(maxsize=1)
def _build():
    mesh = plsc.VectorSubcoreMesh(
        core_axis_name="c", subcore_axis_name="s",
        num_cores=2, num_subcores=_NSUB,
    )
    return pl.kernel(
        _sat_body,
        out_type=jax.ShapeDtypeStruct((_B,), jnp.float32),
        mesh=mesh,
        compiler_params=pltpu.CompilerParams(needs_layout_passes=False),
        scratch_types=[
            pltpu.VMEM((_NV,), jnp.float32),
            pltpu.VMEM((_BUF,), jnp.int32),
            pltpu.VMEM((_BUF,), jnp.int32),
            pltpu.VMEM((_BUF,), jnp.int32),
            pltpu.VMEM((16,), jnp.float32),
            pltpu.VMEM_SHARED((_NSUB, 16), jnp.float32),
            pltpu.VMEM((_NSUB, 16), jnp.float32),
            pltpu.VMEM((_B,), jnp.float32),
            pltpu.SemaphoreType.DMA,
            pltpu.SemaphoreType.DMA,
        ],
    )


def kernel(input, emb_weight, or_weight, clause_idx):
    del input
    packed = lax.shift_left(clause_idx, 1) | (or_weight < 0).astype(jnp.int32)
    return _build()(emb_weight, packed.T.reshape(-1))
